# baseline (speedup 1.0000x reference)
import functools

import jax
import jax.numpy as jnp
from jax import lax
from jax.experimental import pallas as pl
from jax.experimental.pallas import tpu as pltpu
from jax.experimental.pallas import tpu_sc as plsc

_KFAC = 4
_DFAC = 64
_D = _KFAC * _DFAC
_N = 10000
_M = 16
_E = _N * _M
_TAU = 0.1
_ROUTIT = 6
_NITEMS = 1000

_RB = 400
_GC = 200
_NW = 32
_CPW = _E // (_GC * _NW)

_f32 = jnp.float32


def _sc_gather_body(nb_hbm, t_hbm, out_hbm, idx0, idx1, rows0, rows1,
                    sg0, sg1, sw0, sw1):
    wid = lax.axis_index("s") * 2 + lax.axis_index("c")
    base0 = wid * (_GC * _CPW)

    def fetch(i, idx_v, rows_v, sem):
        pltpu.sync_copy(nb_hbm.at[pl.ds(base0 + i * _GC, _GC)], idx_v)
        pltpu.async_copy(t_hbm.at[idx_v], rows_v, sem)

    def wait_fetch(idx_v, rows_v, sem):
        pltpu.make_async_copy(t_hbm.at[idx_v], rows_v, sem).wait()

    def flush(i, rows_v, sem):
        pltpu.async_copy(rows_v, out_hbm.at[pl.ds(base0 + i * _GC, _GC)], sem)

    def wait_flush(i, rows_v, sem):
        pltpu.make_async_copy(
            rows_v, out_hbm.at[pl.ds(base0 + i * _GC, _GC)], sem).wait()

    fetch(0, idx0, rows0, sg0)

    def pair(j, carry):
        a = 2 * j + 1
        b = 2 * j + 2

        @pl.when(j > 0)
        def _():
            wait_flush(a - 2, rows1, sw1)
        fetch(a, idx1, rows1, sg1)
        wait_fetch(idx0, rows0, sg0)
        flush(b - 2, rows0, sw0)
        wait_flush(b - 2, rows0, sw0)
        fetch(b, idx0, rows0, sg0)
        wait_fetch(idx1, rows1, sg1)
        flush(a, rows1, sw1)
        return carry

    lax.fori_loop(0, (_CPW - 1) // 2, pair, 0)
    wait_flush(_CPW - 2, rows1, sw1)
    wait_fetch(idx0, rows0, sg0)
    pltpu.sync_copy(rows0, out_hbm.at[pl.ds(base0 + (_CPW - 1) * _GC, _GC)])


@functools.cache
def _sc_gather_call():
    return functools.partial(
        pl.kernel,
        out_type=jax.ShapeDtypeStruct((_E, _D), _f32),
        mesh=plsc.VectorSubcoreMesh(
            core_axis_name="c", subcore_axis_name="s",
            num_cores=2, num_subcores=16,
        ),
        scratch_types=[
            pltpu.VMEM((_GC,), jnp.int32),
            pltpu.VMEM((_GC,), jnp.int32),
            pltpu.VMEM((_GC, _D), _f32),
            pltpu.VMEM((_GC, _D), _f32),
            pltpu.SemaphoreType.DMA,
            pltpu.SemaphoreType.DMA,
            pltpu.SemaphoreType.DMA,
            pltpu.SemaphoreType.DMA,
        ],
    )(_sc_gather_body)


def _gather(nb, t):
    return _sc_gather_call()(nb, t).reshape(_M, _N, _D)


def _seg_mat():
    row = lax.broadcasted_iota(jnp.int32, (_D, _KFAC), 0) // _DFAC
    col = lax.broadcasted_iota(jnp.int32, (_D, _KFAC), 1)
    return (row == col).astype(_f32)


def _seg_mat_t():
    row = lax.broadcasted_iota(jnp.int32, (_KFAC, _D), 0)
    col = lax.broadcasted_iota(jnp.int32, (_KFAC, _D), 1) // _DFAC
    return (row == col).astype(_f32)


def _seg_pair_mat():
    row = lax.broadcasted_iota(jnp.int32, (_D, _D), 0) // _DFAC
    col = lax.broadcasted_iota(jnp.int32, (_D, _D), 1) // _DFAC
    return (row == col).astype(_f32)


def _mm(a, b):
    return lax.dot_general(a, b, (((1,), (0,)), ((), ())),
                           preferred_element_type=_f32)


def _fnorm_seg(u):
    ns = _mm(u * u, _seg_pair_mat())
    return u / jnp.maximum(jnp.sqrt(ns), 1e-12)


def _route6(g_ref, t):
    S = _seg_mat()
    St = _seg_mat_t()

    def one_iter(u, last):
        unew = t
        for m in range(_M):
            gm = g_ref[m]
            em = jnp.exp(_mm(gm * u, S))
            wm = em / jnp.sum(em, axis=1, keepdims=True)
            unew = unew + gm * _mm(wm, St)
        if not last:
            unew = _fnorm_seg(unew)
        return unew

    u = lax.fori_loop(0, _ROUTIT - 1, lambda i, u: one_iter(u, False), t)
    return one_iter(u, True)


def _prep_body(items_ref, cores_ref, itemsn_ref, catesT_ref):
    it = items_ref[...]
    itn = it / (jnp.sqrt(jnp.sum(it * it, axis=1, keepdims=True)) + 1e-15)
    co = cores_ref[...]
    cn = co / (jnp.sqrt(jnp.sum(co * co, axis=1, keepdims=True)) + 1e-15)
    lg = lax.dot_general(cn, itn, (((1,), (1,)), ((), ())),
                         preferred_element_type=_f32,
                         precision=lax.Precision.HIGHEST) / _TAU
    mx = jnp.max(lg, axis=0, keepdims=True)
    e = jnp.exp(lg - mx)
    catesT_ref[...] = e / jnp.sum(e, axis=0, keepdims=True)
    itemsn_ref[...] = itn


def _enc_body(x_ref, catesT_ref, w0_ref, b0_ref, w1_ref, b1_ref, z_ref, t_ref):
    x = x_ref[...]
    for k in range(_KFAC):
        xk = x * catesT_ref[k:k + 1, :]
        h = xk / (jnp.sqrt(jnp.sum(xk * xk, axis=1, keepdims=True)) + 1e-15)
        h1 = jnp.tanh(
            lax.dot_general(h, w0_ref[...], (((1,), (1,)), ((), ())),
                            preferred_element_type=_f32) + b0_ref[...])
        h2 = lax.dot_general(h1, w1_ref[...], (((1,), (1,)), ((), ())),
                             preferred_element_type=_f32) + b1_ref[...]
        mu = h2[:, :_DFAC]
        zk = mu / (jnp.sqrt(jnp.sum(mu * mu, axis=1, keepdims=True)) + 1e-15)
        z_ref[:, k * _DFAC:(k + 1) * _DFAC] = zk
        tn = jnp.sqrt(jnp.sum(zk * zk, axis=1, keepdims=True))
        t_ref[:, k * _DFAC:(k + 1) * _DFAC] = zk / jnp.maximum(tn, 1e-12)


def _route_mid_body(g_ref, t_ref, wc_ref, bc_ref, out_ref):
    u = _route6(g_ref, t_ref[...])
    a = jnp.maximum(u, 0.0)
    c = lax.dot_general(a, wc_ref[...], (((1,), (1,)), ((), ())),
                        preferred_element_type=_f32) + bc_ref[...]
    out_ref[...] = _fnorm_seg(jnp.maximum(c, 0.0))


def _route_res_body(g_ref, t_ref, z_ref, out_ref):
    u = _route6(g_ref, t_ref[...])
    o = 0.5 * z_ref[...] + jnp.maximum(u, 0.0)
    out_ref[...] = _fnorm_seg(o)


def _route_fin_body(g_ref, t_ref, z_ref, itemsn_ref, catesT_ref, out_ref):
    u = _route6(g_ref, t_ref[...])
    o = 0.5 * z_ref[...] + jnp.maximum(u, 0.0)
    probs = None
    for k in range(_KFAC):
        ok = o[:, k * _DFAC:(k + 1) * _DFAC]
        vk = ok / (jnp.sqrt(jnp.sum(ok * ok, axis=1, keepdims=True)) + 1e-15)
        lk = lax.dot_general(vk, itemsn_ref[...], (((1,), (1,)), ((), ())),
                             preferred_element_type=_f32) / _TAU
        pk = jnp.exp(lk) * catesT_ref[k:k + 1, :]
        probs = pk if probs is None else probs + pk
    lg = jnp.log(probs)
    mx = jnp.max(lg, axis=1, keepdims=True)
    out_ref[...] = lg - mx - jnp.log(
        jnp.sum(jnp.exp(lg - mx), axis=1, keepdims=True))


_GRID = _N // _RB


def _full(shape):
    return pl.BlockSpec(shape, lambda i: (0,) * len(shape))


def _rows(width):
    return pl.BlockSpec((_RB, width), lambda i: (i, 0))


_G_SPEC = pl.BlockSpec((_M, _RB, _D), lambda i: (0, i, 0))


_prep = pl.pallas_call(
    _prep_body,
    out_shape=[jax.ShapeDtypeStruct((_NITEMS, _DFAC), _f32),
               jax.ShapeDtypeStruct((_KFAC, _NITEMS), _f32)],
)

_enc = pl.pallas_call(
    _enc_body,
    grid=(_GRID,),
    in_specs=[_rows(_NITEMS), _full((_KFAC, _NITEMS)), _full((_DFAC, _NITEMS)),
              _full((1, _DFAC)), _full((2 * _DFAC, _DFAC)), _full((1, 2 * _DFAC))],
    out_specs=[_rows(_D), _rows(_D)],
    out_shape=[jax.ShapeDtypeStruct((_N, _D), _f32),
               jax.ShapeDtypeStruct((_N, _D), _f32)],
)

_route_mid = pl.pallas_call(
    _route_mid_body,
    grid=(_GRID,),
    in_specs=[_G_SPEC, _rows(_D), _full((_D, _D)), _full((1, _D))],
    out_specs=_rows(_D),
    out_shape=jax.ShapeDtypeStruct((_N, _D), _f32),
)

_route_res = pl.pallas_call(
    _route_res_body,
    grid=(_GRID,),
    in_specs=[_G_SPEC, _rows(_D), _rows(_D)],
    out_specs=_rows(_D),
    out_shape=jax.ShapeDtypeStruct((_N, _D), _f32),
)

_route_fin = pl.pallas_call(
    _route_fin_body,
    grid=(_GRID,),
    in_specs=[_G_SPEC, _rows(_D), _rows(_D), _full((_NITEMS, _DFAC)),
              _full((_KFAC, _NITEMS))],
    out_specs=_rows(_NITEMS),
    out_shape=jax.ShapeDtypeStruct((_N, _NITEMS), _f32),
)


def kernel(save_emb, neighbors, input_ph, is_training_ph, anneal_ph,
           items, cores, W0, b0, W1, b1, Wc, bc):
    nb = neighbors.reshape(_N, _M).T.reshape(-1).astype(jnp.int32)
    itemsn, catesT = _prep(items, cores)
    z, t = _enc(input_ph, catesT, W0, b0.reshape(1, -1), W1, b1.reshape(1, -1))
    bc2 = bc.reshape(1, -1)
    t = _route_mid(_gather(nb, t), t, Wc, bc2)
    t = _route_res(_gather(nb, t), t, z)
    t = _route_mid(_gather(nb, t), t, Wc, bc2)
    return _route_fin(_gather(nb, t), t, z, itemsn, catesT)


# speedup vs baseline: 1.0819x; 1.0819x over previous
import functools

import jax
import jax.numpy as jnp
from jax import lax
from jax.experimental import pallas as pl
from jax.experimental.pallas import tpu as pltpu
from jax.experimental.pallas import tpu_sc as plsc

_KFAC = 4
_DFAC = 64
_D = _KFAC * _DFAC
_N = 10000
_M = 16
_E = _N * _M
_TAU = 0.1
_ROUTIT = 6
_NITEMS = 1000

_RB = 400
_GC = 200
_NW = 32
_CPW = _E // (_GC * _NW)

_f32 = jnp.float32


def _sc_gather_body(nb_hbm, t_hbm, out_hbm, idx0, idx1, rows0, rows1,
                    sg0, sg1, sw0, sw1):
    wid = lax.axis_index("s") * 2 + lax.axis_index("c")
    base0 = wid * (_GC * _CPW)

    def fetch(i, idx_v, rows_v, sem):
        pltpu.sync_copy(nb_hbm.at[pl.ds(base0 + i * _GC, _GC)], idx_v)
        pltpu.async_copy(t_hbm.at[idx_v], rows_v, sem)

    def wait_fetch(idx_v, rows_v, sem):
        pltpu.make_async_copy(t_hbm.at[idx_v], rows_v, sem).wait()

    def flush(i, rows_v, sem):
        pltpu.async_copy(rows_v, out_hbm.at[pl.ds(base0 + i * _GC, _GC)], sem)

    def wait_flush(i, rows_v, sem):
        pltpu.make_async_copy(
            rows_v, out_hbm.at[pl.ds(base0 + i * _GC, _GC)], sem).wait()

    fetch(0, idx0, rows0, sg0)

    def pair(j, carry):
        a = 2 * j + 1
        b = 2 * j + 2

        @pl.when(j > 0)
        def _():
            wait_flush(a - 2, rows1, sw1)
        fetch(a, idx1, rows1, sg1)
        wait_fetch(idx0, rows0, sg0)
        flush(b - 2, rows0, sw0)
        wait_flush(b - 2, rows0, sw0)
        fetch(b, idx0, rows0, sg0)
        wait_fetch(idx1, rows1, sg1)
        flush(a, rows1, sw1)
        return carry

    lax.fori_loop(0, (_CPW - 1) // 2, pair, 0)
    wait_flush(_CPW - 2, rows1, sw1)
    wait_fetch(idx0, rows0, sg0)
    pltpu.sync_copy(rows0, out_hbm.at[pl.ds(base0 + (_CPW - 1) * _GC, _GC)])


@functools.cache
def _sc_gather_call():
    return functools.partial(
        pl.kernel,
        out_type=jax.ShapeDtypeStruct((_E, _D), _f32),
        mesh=plsc.VectorSubcoreMesh(
            core_axis_name="c", subcore_axis_name="s",
            num_cores=2, num_subcores=16,
        ),
        scratch_types=[
            pltpu.VMEM((_GC,), jnp.int32),
            pltpu.VMEM((_GC,), jnp.int32),
            pltpu.VMEM((_GC, _D), _f32),
            pltpu.VMEM((_GC, _D), _f32),
            pltpu.SemaphoreType.DMA,
            pltpu.SemaphoreType.DMA,
            pltpu.SemaphoreType.DMA,
            pltpu.SemaphoreType.DMA,
        ],
    )(_sc_gather_body)


def _gather(nb, t):
    return _sc_gather_call()(nb, t).reshape(_M, _N, _D)


def _seg_mat():
    row = lax.broadcasted_iota(jnp.int32, (_D, _KFAC), 0) // _DFAC
    col = lax.broadcasted_iota(jnp.int32, (_D, _KFAC), 1)
    return (row == col).astype(_f32)


def _seg_mat_t():
    row = lax.broadcasted_iota(jnp.int32, (_KFAC, _D), 0)
    col = lax.broadcasted_iota(jnp.int32, (_KFAC, _D), 1) // _DFAC
    return (row == col).astype(_f32)


def _seg_pair_mat():
    row = lax.broadcasted_iota(jnp.int32, (_D, _D), 0) // _DFAC
    col = lax.broadcasted_iota(jnp.int32, (_D, _D), 1) // _DFAC
    return (row == col).astype(_f32)


def _mm(a, b):
    return lax.dot_general(a, b, (((1,), (0,)), ((), ())),
                           preferred_element_type=_f32)


def _fnorm_seg(u):
    ns = _mm(u * u, _seg_pair_mat())
    return u / jnp.maximum(jnp.sqrt(ns), 1e-12)


def _route6(g_ref, t):
    S = _seg_mat()
    St = _seg_mat_t()

    def one_iter(u, last):
        ws = []
        for m in range(_M):
            em = jnp.exp(_mm(g_ref[m] * u, S))
            ws.append(em / jnp.sum(em, axis=1, keepdims=True))
        unew = t
        for m in range(_M):
            unew = unew + g_ref[m] * _mm(ws[m], St)
        if not last:
            unew = _fnorm_seg(unew)
        return unew

    u = lax.fori_loop(0, _ROUTIT - 1, lambda i, u: one_iter(u, False), t)
    return one_iter(u, True)


def _prep_body(items_ref, cores_ref, itemsn_ref, catesT_ref):
    it = items_ref[...]
    itn = it / (jnp.sqrt(jnp.sum(it * it, axis=1, keepdims=True)) + 1e-15)
    co = cores_ref[...]
    cn = co / (jnp.sqrt(jnp.sum(co * co, axis=1, keepdims=True)) + 1e-15)
    lg = lax.dot_general(cn, itn, (((1,), (1,)), ((), ())),
                         preferred_element_type=_f32,
                         precision=lax.Precision.HIGHEST) / _TAU
    mx = jnp.max(lg, axis=0, keepdims=True)
    e = jnp.exp(lg - mx)
    catesT_ref[...] = e / jnp.sum(e, axis=0, keepdims=True)
    itemsn_ref[...] = itn


def _enc_body(x_ref, catesT_ref, w0_ref, b0_ref, w1_ref, b1_ref, z_ref, t_ref):
    x = x_ref[...]
    for k in range(_KFAC):
        xk = x * catesT_ref[k:k + 1, :]
        h = xk / (jnp.sqrt(jnp.sum(xk * xk, axis=1, keepdims=True)) + 1e-15)
        h1 = jnp.tanh(
            lax.dot_general(h, w0_ref[...], (((1,), (1,)), ((), ())),
                            preferred_element_type=_f32) + b0_ref[...])
        h2 = lax.dot_general(h1, w1_ref[...], (((1,), (1,)), ((), ())),
                             preferred_element_type=_f32) + b1_ref[...]
        mu = h2[:, :_DFAC]
        zk = mu / (jnp.sqrt(jnp.sum(mu * mu, axis=1, keepdims=True)) + 1e-15)
        z_ref[:, k * _DFAC:(k + 1) * _DFAC] = zk
        tn = jnp.sqrt(jnp.sum(zk * zk, axis=1, keepdims=True))
        t_ref[:, k * _DFAC:(k + 1) * _DFAC] = zk / jnp.maximum(tn, 1e-12)


def _route_mid_body(g_ref, t_ref, wc_ref, bc_ref, out_ref):
    u = _route6(g_ref, t_ref[...])
    a = jnp.maximum(u, 0.0)
    c = lax.dot_general(a, wc_ref[...], (((1,), (1,)), ((), ())),
                        preferred_element_type=_f32) + bc_ref[...]
    out_ref[...] = _fnorm_seg(jnp.maximum(c, 0.0))


def _route_res_body(g_ref, t_ref, z_ref, out_ref):
    u = _route6(g_ref, t_ref[...])
    o = 0.5 * z_ref[...] + jnp.maximum(u, 0.0)
    out_ref[...] = _fnorm_seg(o)


def _route_fin_body(g_ref, t_ref, z_ref, itemsn_ref, catesT_ref, out_ref):
    u = _route6(g_ref, t_ref[...])
    o = 0.5 * z_ref[...] + jnp.maximum(u, 0.0)
    probs = None
    for k in range(_KFAC):
        ok = o[:, k * _DFAC:(k + 1) * _DFAC]
        vk = ok / (jnp.sqrt(jnp.sum(ok * ok, axis=1, keepdims=True)) + 1e-15)
        lk = lax.dot_general(vk, itemsn_ref[...], (((1,), (1,)), ((), ())),
                             preferred_element_type=_f32) / _TAU
        pk = jnp.exp(lk) * catesT_ref[k:k + 1, :]
        probs = pk if probs is None else probs + pk
    lg = jnp.log(probs)
    mx = jnp.max(lg, axis=1, keepdims=True)
    out_ref[...] = lg - mx - jnp.log(
        jnp.sum(jnp.exp(lg - mx), axis=1, keepdims=True))


_GRID = _N // _RB


def _full(shape):
    return pl.BlockSpec(shape, lambda i: (0,) * len(shape))


def _rows(width):
    return pl.BlockSpec((_RB, width), lambda i: (i, 0))


_G_SPEC = pl.BlockSpec((_M, _RB, _D), lambda i: (0, i, 0))


_prep = pl.pallas_call(
    _prep_body,
    out_shape=[jax.ShapeDtypeStruct((_NITEMS, _DFAC), _f32),
               jax.ShapeDtypeStruct((_KFAC, _NITEMS), _f32)],
)

_enc = pl.pallas_call(
    _enc_body,
    grid=(_GRID,),
    in_specs=[_rows(_NITEMS), _full((_KFAC, _NITEMS)), _full((_DFAC, _NITEMS)),
              _full((1, _DFAC)), _full((2 * _DFAC, _DFAC)), _full((1, 2 * _DFAC))],
    out_specs=[_rows(_D), _rows(_D)],
    out_shape=[jax.ShapeDtypeStruct((_N, _D), _f32),
               jax.ShapeDtypeStruct((_N, _D), _f32)],
)

_route_mid = pl.pallas_call(
    _route_mid_body,
    grid=(_GRID,),
    in_specs=[_G_SPEC, _rows(_D), _full((_D, _D)), _full((1, _D))],
    out_specs=_rows(_D),
    out_shape=jax.ShapeDtypeStruct((_N, _D), _f32),
)

_route_res = pl.pallas_call(
    _route_res_body,
    grid=(_GRID,),
    in_specs=[_G_SPEC, _rows(_D), _rows(_D)],
    out_specs=_rows(_D),
    out_shape=jax.ShapeDtypeStruct((_N, _D), _f32),
)

_route_fin = pl.pallas_call(
    _route_fin_body,
    grid=(_GRID,),
    in_specs=[_G_SPEC, _rows(_D), _rows(_D), _full((_NITEMS, _DFAC)),
              _full((_KFAC, _NITEMS))],
    out_specs=_rows(_NITEMS),
    out_shape=jax.ShapeDtypeStruct((_N, _NITEMS), _f32),
)


def kernel(save_emb, neighbors, input_ph, is_training_ph, anneal_ph,
           items, cores, W0, b0, W1, b1, Wc, bc):
    nb = neighbors.reshape(_N, _M).T.reshape(-1).astype(jnp.int32)
    itemsn, catesT = _prep(items, cores)
    z, t = _enc(input_ph, catesT, W0, b0.reshape(1, -1), W1, b1.reshape(1, -1))
    bc2 = bc.reshape(1, -1)
    t = _route_mid(_gather(nb, t), t, Wc, bc2)
    t = _route_res(_gather(nb, t), t, z)
    t = _route_mid(_gather(nb, t), t, Wc, bc2)
    return _route_fin(_gather(nb, t), t, z, itemsn, catesT)
